# Initial kernel scaffold; baseline (speedup 1.0000x reference)
import jax
import jax.numpy as jnp
from jax.experimental import pallas as pl


def kernel(x, edge_index, Wu1, Wg1, Wg2, Wsu, Wsg, Wst, Wsl, W1, W2, W3, gamma1, beta1, gamma2, beta2, gamma3, beta3, gamma4, beta4, gamma5, beta5, gamma6, beta6, gamma7, beta7, bsl, b1, b2, b3):
    raise NotImplementedError("write your pallas kernel here")


# speedup vs baseline: 7.2097x; 7.2097x over previous
import functools

import jax
import jax.numpy as jnp
from jax import lax
from jax.experimental import pallas as pl
from jax.experimental.pallas import tpu as pltpu
from jax.experimental.pallas import tpu_sc as plsc

N = 10000
E = 320000
C = 128
EPS = 1e-5

_NC = 2
_NS = 16
_NW = _NC * _NS
_K = 128
_EPT = E // _NW
_NCHUNK = (_EPT + _K - 1) // _K
_EPAD = _NCHUNK * _K - _EPT
_NPAD = 10240
_RPT = _NPAD // _NS
_ZB = _RPT // _K


def _sc_body(h_hbm, src_hbm, dst_hbm, out_hbm, acc_shared, src_v, dst_v,
             rows_v, sem):
    cid = lax.axis_index("c")
    sid = lax.axis_index("s")
    wid = sid * _NC + cid

    z = jnp.zeros((16,), jnp.float32)

    def zrow(i, carry):
        for j in range(C // 16):
            rows_v[i, pl.ds(j * 16, 16)] = z
        return carry

    lax.fori_loop(0, _K, zrow, 0)
    for t in range(_ZB):
        pltpu.sync_copy(rows_v, acc_shared.at[pl.ds(sid * _RPT + t * _K, _K)])

    pltpu.sync_copy(src_hbm.at[wid], src_v)
    pltpu.sync_copy(dst_hbm.at[wid], dst_v)

    plsc.subcore_barrier()

    def step(j, carry):
        pltpu.async_copy(h_hbm.at[src_v.at[j]], rows_v, sem).wait()
        pltpu.sync_copy(rows_v, acc_shared.at[dst_v.at[j]], add=True)
        return carry

    lax.fori_loop(0, _NCHUNK, step, 0)

    plsc.subcore_barrier()

    pltpu.sync_copy(acc_shared.at[pl.ds(sid * _RPT, _RPT)],
                    out_hbm.at[cid, pl.ds(sid * _RPT, _RPT)])


_sc_scatter = functools.partial(
    pl.kernel,
    _sc_body,
    out_type=jax.ShapeDtypeStruct((_NC, _NPAD, C), jnp.float32),
    mesh=plsc.VectorSubcoreMesh(core_axis_name="c", subcore_axis_name="s"),
    scratch_types=[
        pltpu.VMEM_SHARED((_NPAD, C), jnp.float32),
        pltpu.VMEM((_NCHUNK, _K), jnp.int32),
        pltpu.VMEM((_NCHUNK, _K), jnp.int32),
        pltpu.VMEM((_K, C), jnp.float32),
        pltpu.SemaphoreType.DMA,
    ],
)()


_PREC = jax.lax.Precision.HIGHEST


def _dot(a, b):
    return jax.lax.dot_general(a, b, (((1,), (0,)), ((), ())),
                               preferred_element_type=jnp.float32,
                               precision=_PREC)


def _conv_body(x_ref, p_ref, w_ref, g_ref, b_ref, o_ref):
    x = x_ref[...]
    agg = p_ref[0, :N] + p_ref[1, :N]
    h = _dot(x, w_ref[:C]) + _dot(agg, w_ref[C:])
    mu = jnp.mean(h, axis=0, keepdims=True)
    d = h - mu
    var = jnp.mean(d * d, axis=0, keepdims=True)
    o_ref[...] = jnp.maximum(
        d * jax.lax.rsqrt(var + EPS) * g_ref[...] + b_ref[...], 0.0)


def _conv_tc(x, p, w, gamma, beta):
    return pl.pallas_call(
        _conv_body,
        out_shape=jax.ShapeDtypeStruct((N, C), jnp.float32),
    )(x, p, w, gamma, beta)


def _tail_body(h_ref, wsl_ref, bsl_ref, g_ref, b_ref, w1_ref, b1_ref,
               w2_ref, b2_ref, w3_ref, b3_ref, o_ref):
    h = h_ref[...]
    t = jnp.maximum(_dot(h, wsl_ref[...]) + bsl_ref[...], 0.0)
    mu = jnp.mean(t, axis=0, keepdims=True)
    d = t - mu
    var = jnp.mean(d * d, axis=0, keepdims=True)
    t = jnp.maximum(d * jax.lax.rsqrt(var + EPS) * g_ref[...] + b_ref[...],
                    0.0)
    pm = jnp.mean(t, axis=0, keepdims=True)
    px = jnp.max(t, axis=0, keepdims=True)
    o = jnp.maximum(_dot(pm, w1_ref[:C]) + _dot(px, w1_ref[C:])
                    + b1_ref[...], 0.0)
    o = jnp.maximum(_dot(o, w2_ref[...]) + b2_ref[...], 0.0)
    o_ref[...] = _dot(o, w3_ref[...]) + b3_ref[...]


def _tail_tc(h, wsl, bsl, gamma, beta, w1, b1, w2, b2, w3, b3):
    return pl.pallas_call(
        _tail_body,
        out_shape=jax.ShapeDtypeStruct((1, 1), jnp.float32),
    )(h, wsl, bsl, gamma, beta, w1, b1, w2, b2, w3, b3)


def _pad_edges(edge_index):
    srcs = edge_index[0].reshape(_NW, _EPT)
    dsts = edge_index[1].reshape(_NW, _EPT)
    wids = jnp.arange(_NW, dtype=jnp.int32)[:, None]
    ks = jnp.arange(_EPAD, dtype=jnp.int32)[None, :]
    pad_src = (wids * 313 + ks * 89) % N
    pad_dst = N + (wids * 7 + ks) % (_NPAD - N)
    src_r = jnp.concatenate([srcs, pad_src], axis=1).reshape(
        _NW, _NCHUNK, _K)
    dst_r = jnp.concatenate([dsts, pad_dst], axis=1).reshape(
        _NW, _NCHUNK, _K)
    return src_r, dst_r


def kernel(x, edge_index, Wu1, Wg1, Wg2, Wsu, Wsg, Wst, Wsl, W1, W2, W3,
           gamma1, beta1, gamma2, beta2, gamma3, beta3, gamma4, beta4,
           gamma5, beta5, gamma6, beta6, gamma7, beta7, bsl, b1, b2, b3):
    src_r, dst_r = _pad_edges(edge_index)

    h = x
    layers = ((Wu1, gamma1, beta1), (Wg1, gamma2, beta2),
              (Wg2, gamma3, beta3), (Wsu, gamma4, beta4),
              (Wsg, gamma5, beta5), (Wst, gamma6, beta6))
    for w, g, b in layers:
        p = _sc_scatter(h, src_r, dst_r)
        h = _conv_tc(h, p, w, g.reshape(1, C), b.reshape(1, C))

    return _tail_tc(h, Wsl, bsl.reshape(1, C), gamma7.reshape(1, C),
                    beta7.reshape(1, C), W1, b1.reshape(1, C), W2,
                    b2.reshape(1, C), W3, b3.reshape(1, 1))


# speedup vs baseline: 11.0023x; 1.5260x over previous
import functools

import jax
import jax.numpy as jnp
from jax import lax
from jax.experimental import pallas as pl
from jax.experimental.pallas import tpu as pltpu
from jax.experimental.pallas import tpu_sc as plsc

N = 10000
E = 320000
C = 128
EPS = 1e-5

_NC = 2
_NS = 16
_NW = _NC * _NS
_K = 128
_EPT = E // _NW
_NCHUNK = 80
_NPAIR = _NCHUNK // 2
_EPAD = _NCHUNK * _K - _EPT
_NPAD = 10112
_RPT = _NPAD // _NS


def _sc_body(h_hbm, src_hbm, dst_hbm, out_hbm, acc_shared, src_v, dst_ring,
             rows_v, sem0, sem1, semd0, semd1):
    cid = lax.axis_index("c")
    sid = lax.axis_index("s")
    wid = sid * _NC + cid

    ci = pltpu.async_copy(src_hbm.at[wid], src_v, sem0)
    pltpu.async_copy(dst_hbm.at[wid, 0], dst_ring.at[0], semd0)
    pltpu.async_copy(dst_hbm.at[wid, 1], dst_ring.at[1], semd1)

    z = jnp.zeros((16,), jnp.float32)

    def zrow(i, carry):
        for j in range(C // 16):
            rows_v[0, i, pl.ds(j * 16, 16)] = z
        return carry

    lax.fori_loop(0, _K, zrow, 0)
    zcs = [pltpu.async_copy(rows_v.at[0],
                            acc_shared.at[pl.ds(sid * _RPT + t * _K, _K)],
                            sem1)
           for t in range(4)]
    zcs.append(pltpu.async_copy(
        rows_v.at[0, pl.ds(0, _RPT - 4 * _K)],
        acc_shared.at[pl.ds(sid * _RPT + 4 * _K, _RPT - 4 * _K)], sem1))
    for c in zcs:
        c.wait()
    ci.wait()

    plsc.subcore_barrier()

    pltpu.async_copy(h_hbm.at[src_v.at[0]], rows_v.at[0], sem0)

    def pair(t, carry):
        j0 = 2 * t
        j1 = j0 + 1
        pltpu.async_copy(h_hbm.at[src_v.at[j1]], rows_v.at[1], sem1)
        pltpu.make_async_copy(h_hbm.at[src_v.at[j0]], rows_v.at[0],
                              sem0).wait()
        pltpu.make_async_copy(dst_hbm.at[wid, j0], dst_ring.at[0],
                              semd0).wait()
        pltpu.sync_copy(rows_v.at[0], acc_shared.at[dst_ring.at[0]],
                        add=True)

        @pl.when(t + 1 < _NPAIR)
        def _():
            pltpu.async_copy(dst_hbm.at[wid, j0 + 2], dst_ring.at[0], semd0)
            pltpu.async_copy(h_hbm.at[src_v.at[j0 + 2]], rows_v.at[0], sem0)

        pltpu.make_async_copy(h_hbm.at[src_v.at[j1]], rows_v.at[1],
                              sem1).wait()
        pltpu.make_async_copy(dst_hbm.at[wid, j1], dst_ring.at[1],
                              semd1).wait()
        pltpu.sync_copy(rows_v.at[1], acc_shared.at[dst_ring.at[1]],
                        add=True)

        @pl.when(t + 1 < _NPAIR)
        def _():
            pltpu.async_copy(dst_hbm.at[wid, j1 + 2], dst_ring.at[1], semd1)

        return carry

    lax.fori_loop(0, _NPAIR, pair, 0)

    plsc.subcore_barrier()

    pltpu.sync_copy(acc_shared.at[pl.ds(sid * _RPT, _RPT)],
                    out_hbm.at[cid, pl.ds(sid * _RPT, _RPT)])


_sc_scatter = functools.partial(
    pl.kernel,
    _sc_body,
    out_type=jax.ShapeDtypeStruct((_NC, _NPAD, C), jnp.float32),
    mesh=plsc.VectorSubcoreMesh(core_axis_name="c", subcore_axis_name="s"),
    scratch_types=[
        pltpu.VMEM_SHARED((_NPAD, C), jnp.float32),
        pltpu.VMEM((_NCHUNK, _K), jnp.int32),
        pltpu.VMEM((2, _K), jnp.int32),
        pltpu.VMEM((2, _K, C), jnp.float32),
        pltpu.SemaphoreType.DMA,
        pltpu.SemaphoreType.DMA,
        pltpu.SemaphoreType.DMA,
        pltpu.SemaphoreType.DMA,
    ],
)()


_PREC = jax.lax.Precision.HIGHEST


def _dot(a, b):
    return jax.lax.dot_general(a, b, (((1,), (0,)), ((), ())),
                               preferred_element_type=jnp.float32,
                               precision=_PREC)


def _conv_body(x_ref, p_ref, w_ref, g_ref, b_ref, o_ref):
    x = x_ref[...]
    agg = p_ref[0, :N] + p_ref[1, :N]
    h = _dot(x, w_ref[:C]) + _dot(agg, w_ref[C:])
    mu = jnp.mean(h, axis=0, keepdims=True)
    d = h - mu
    var = jnp.mean(d * d, axis=0, keepdims=True)
    o_ref[...] = jnp.maximum(
        d * jax.lax.rsqrt(var + EPS) * g_ref[...] + b_ref[...], 0.0)


def _conv_tc(x, p, w, gamma, beta):
    return pl.pallas_call(
        _conv_body,
        out_shape=jax.ShapeDtypeStruct((N, C), jnp.float32),
    )(x, p, w, gamma, beta)


def _tail_body(h_ref, wsl_ref, bsl_ref, g_ref, b_ref, w1_ref, b1_ref,
               w2_ref, b2_ref, w3_ref, b3_ref, o_ref):
    h = h_ref[...]
    t = jnp.maximum(_dot(h, wsl_ref[...]) + bsl_ref[...], 0.0)
    mu = jnp.mean(t, axis=0, keepdims=True)
    d = t - mu
    var = jnp.mean(d * d, axis=0, keepdims=True)
    t = jnp.maximum(d * jax.lax.rsqrt(var + EPS) * g_ref[...] + b_ref[...],
                    0.0)
    pm = jnp.mean(t, axis=0, keepdims=True)
    px = jnp.max(t, axis=0, keepdims=True)
    o = jnp.maximum(_dot(pm, w1_ref[:C]) + _dot(px, w1_ref[C:])
                    + b1_ref[...], 0.0)
    o = jnp.maximum(_dot(o, w2_ref[...]) + b2_ref[...], 0.0)
    o_ref[...] = _dot(o, w3_ref[...]) + b3_ref[...]


def _tail_tc(h, wsl, bsl, gamma, beta, w1, b1, w2, b2, w3, b3):
    return pl.pallas_call(
        _tail_body,
        out_shape=jax.ShapeDtypeStruct((1, 1), jnp.float32),
    )(h, wsl, bsl, gamma, beta, w1, b1, w2, b2, w3, b3)


def _pad_edges(edge_index):
    srcs = edge_index[0].reshape(_NW, _EPT)
    dsts = edge_index[1].reshape(_NW, _EPT)
    wids = jnp.arange(_NW, dtype=jnp.int32)[:, None]
    ks = jnp.arange(_EPAD, dtype=jnp.int32)[None, :]
    pad_src = (wids * 313 + ks * 89) % N
    pad_dst = N + (wids * 7 + ks) % (_NPAD - N)
    src_r = jnp.concatenate([srcs, pad_src], axis=1).reshape(
        _NW, _NCHUNK, _K)
    dst_r = jnp.concatenate([dsts, pad_dst], axis=1).reshape(
        _NW, _NCHUNK, _K)
    return src_r, dst_r


def kernel(x, edge_index, Wu1, Wg1, Wg2, Wsu, Wsg, Wst, Wsl, W1, W2, W3,
           gamma1, beta1, gamma2, beta2, gamma3, beta3, gamma4, beta4,
           gamma5, beta5, gamma6, beta6, gamma7, beta7, bsl, b1, b2, b3):
    src_r, dst_r = _pad_edges(edge_index)

    h = x
    layers = ((Wu1, gamma1, beta1), (Wg1, gamma2, beta2),
              (Wg2, gamma3, beta3), (Wsu, gamma4, beta4),
              (Wsg, gamma5, beta5), (Wst, gamma6, beta6))
    for w, g, b in layers:
        p = _sc_scatter(h, src_r, dst_r)
        h = _conv_tc(h, p, w, g.reshape(1, C), b.reshape(1, C))

    return _tail_tc(h, Wsl, bsl.reshape(1, C), gamma7.reshape(1, C),
                    beta7.reshape(1, C), W1, b1.reshape(1, C), W2,
                    b2.reshape(1, C), W3, b3.reshape(1, 1))


# speedup vs baseline: 11.1820x; 1.0163x over previous
import functools

import jax
import jax.numpy as jnp
from jax import lax
from jax.experimental import pallas as pl
from jax.experimental.pallas import tpu as pltpu
from jax.experimental.pallas import tpu_sc as plsc

N = 10000
E = 320000
C = 128
EPS = 1e-5

_NC = 2
_NS = 16
_NW = _NC * _NS
_K = 128
_EPT = E // _NW
_NCHUNK = 80
_NPAIR = _NCHUNK // 2
_EPAD = _NCHUNK * _K - _EPT
_NPAD = 10112
_RPT = _NPAD // _NS


def _sc_body(h_hbm, src_hbm, dst_hbm, out_hbm, acc_shared, src_v, dst_ring,
             rows_v, sem0, sem1, semd0, semd1):
    cid = lax.axis_index("c")
    sid = lax.axis_index("s")
    wid = sid * _NC + cid

    ci = pltpu.async_copy(src_hbm.at[wid], src_v, sem0)
    pltpu.async_copy(dst_hbm.at[wid, 0], dst_ring.at[0], semd0)
    pltpu.async_copy(dst_hbm.at[wid, 1], dst_ring.at[1], semd1)

    z = jnp.zeros((16,), jnp.float32)

    def zrow(i, carry):
        for j in range(C // 16):
            rows_v[1, i, pl.ds(j * 16, 16)] = z
        return carry

    lax.fori_loop(0, _K, zrow, 0)
    zcs = [pltpu.async_copy(rows_v.at[1],
                            acc_shared.at[pl.ds(sid * _RPT + t * _K, _K)],
                            sem1)
           for t in range(4)]
    zcs.append(pltpu.async_copy(
        rows_v.at[1, pl.ds(0, _RPT - 4 * _K)],
        acc_shared.at[pl.ds(sid * _RPT + 4 * _K, _RPT - 4 * _K)], sem1))
    ci.wait()
    pltpu.async_copy(h_hbm.at[src_v.at[0]], rows_v.at[0], sem0)
    for c in zcs:
        c.wait()

    plsc.subcore_barrier()

    def pair(t, carry):
        j0 = 2 * t
        j1 = j0 + 1
        pltpu.async_copy(h_hbm.at[src_v.at[j1]], rows_v.at[1], sem1)
        pltpu.make_async_copy(h_hbm.at[src_v.at[j0]], rows_v.at[0],
                              sem0).wait()
        pltpu.make_async_copy(dst_hbm.at[wid, j0], dst_ring.at[0],
                              semd0).wait()
        pltpu.sync_copy(rows_v.at[0], acc_shared.at[dst_ring.at[0]],
                        add=True)

        @pl.when(t + 1 < _NPAIR)
        def _():
            pltpu.async_copy(dst_hbm.at[wid, j0 + 2], dst_ring.at[0], semd0)
            pltpu.async_copy(h_hbm.at[src_v.at[j0 + 2]], rows_v.at[0], sem0)

        pltpu.make_async_copy(h_hbm.at[src_v.at[j1]], rows_v.at[1],
                              sem1).wait()
        pltpu.make_async_copy(dst_hbm.at[wid, j1], dst_ring.at[1],
                              semd1).wait()
        pltpu.sync_copy(rows_v.at[1], acc_shared.at[dst_ring.at[1]],
                        add=True)

        @pl.when(t + 1 < _NPAIR)
        def _():
            pltpu.async_copy(dst_hbm.at[wid, j1 + 2], dst_ring.at[1], semd1)

        return carry

    lax.fori_loop(0, _NPAIR, pair, 0)

    plsc.subcore_barrier()

    pltpu.sync_copy(acc_shared.at[pl.ds(sid * _RPT, _RPT)],
                    out_hbm.at[cid, pl.ds(sid * _RPT, _RPT)])


_sc_scatter = functools.partial(
    pl.kernel,
    _sc_body,
    out_type=jax.ShapeDtypeStruct((_NC, _NPAD, C), jnp.float32),
    mesh=plsc.VectorSubcoreMesh(core_axis_name="c", subcore_axis_name="s"),
    scratch_types=[
        pltpu.VMEM_SHARED((_NPAD, C), jnp.float32),
        pltpu.VMEM((_NCHUNK, _K), jnp.int32),
        pltpu.VMEM((2, _K), jnp.int32),
        pltpu.VMEM((2, _K, C), jnp.float32),
        pltpu.SemaphoreType.DMA,
        pltpu.SemaphoreType.DMA,
        pltpu.SemaphoreType.DMA,
        pltpu.SemaphoreType.DMA,
    ],
)()


_PREC = jax.lax.Precision.HIGHEST


def _dot(a, b):
    return jax.lax.dot_general(a, b, (((1,), (0,)), ((), ())),
                               preferred_element_type=jnp.float32,
                               precision=_PREC)


def _conv_body(x_ref, p_ref, w_ref, g_ref, b_ref, o_ref):
    x = x_ref[...]
    agg = p_ref[0, :N] + p_ref[1, :N]
    h = _dot(x, w_ref[:C]) + _dot(agg, w_ref[C:])
    mu = jnp.mean(h, axis=0, keepdims=True)
    d = h - mu
    var = jnp.mean(d * d, axis=0, keepdims=True)
    o_ref[...] = jnp.maximum(
        d * jax.lax.rsqrt(var + EPS) * g_ref[...] + b_ref[...], 0.0)


def _conv_tc(x, p, w, gamma, beta):
    return pl.pallas_call(
        _conv_body,
        out_shape=jax.ShapeDtypeStruct((N, C), jnp.float32),
    )(x, p, w, gamma, beta)


def _tail_body(h_ref, wsl_ref, bsl_ref, g_ref, b_ref, w1_ref, b1_ref,
               w2_ref, b2_ref, w3_ref, b3_ref, o_ref):
    h = h_ref[...]
    t = jnp.maximum(_dot(h, wsl_ref[...]) + bsl_ref[...], 0.0)
    mu = jnp.mean(t, axis=0, keepdims=True)
    d = t - mu
    var = jnp.mean(d * d, axis=0, keepdims=True)
    t = jnp.maximum(d * jax.lax.rsqrt(var + EPS) * g_ref[...] + b_ref[...],
                    0.0)
    pm = jnp.mean(t, axis=0, keepdims=True)
    px = jnp.max(t, axis=0, keepdims=True)
    o = jnp.maximum(_dot(pm, w1_ref[:C]) + _dot(px, w1_ref[C:])
                    + b1_ref[...], 0.0)
    o = jnp.maximum(_dot(o, w2_ref[...]) + b2_ref[...], 0.0)
    o_ref[...] = _dot(o, w3_ref[...]) + b3_ref[...]


def _tail_tc(h, wsl, bsl, gamma, beta, w1, b1, w2, b2, w3, b3):
    return pl.pallas_call(
        _tail_body,
        out_shape=jax.ShapeDtypeStruct((1, 1), jnp.float32),
    )(h, wsl, bsl, gamma, beta, w1, b1, w2, b2, w3, b3)


def _pad_edges(edge_index):
    srcs = edge_index[0].reshape(_NW, _EPT)
    dsts = edge_index[1].reshape(_NW, _EPT)
    wids = jnp.arange(_NW, dtype=jnp.int32)[:, None]
    ks = jnp.arange(_EPAD, dtype=jnp.int32)[None, :]
    pad_src = (wids * 313 + ks * 89) % N
    pad_dst = N + (wids * 7 + ks) % (_NPAD - N)
    src_r = jnp.concatenate([srcs, pad_src], axis=1).reshape(
        _NW, _NCHUNK, _K)
    dst_r = jnp.concatenate([dsts, pad_dst], axis=1).reshape(
        _NW, _NCHUNK, _K)
    return src_r, dst_r


def kernel(x, edge_index, Wu1, Wg1, Wg2, Wsu, Wsg, Wst, Wsl, W1, W2, W3,
           gamma1, beta1, gamma2, beta2, gamma3, beta3, gamma4, beta4,
           gamma5, beta5, gamma6, beta6, gamma7, beta7, bsl, b1, b2, b3):
    src_r, dst_r = _pad_edges(edge_index)

    h = x
    layers = ((Wu1, gamma1, beta1), (Wg1, gamma2, beta2),
              (Wg2, gamma3, beta3), (Wsu, gamma4, beta4),
              (Wsg, gamma5, beta5), (Wst, gamma6, beta6))
    for w, g, b in layers:
        p = _sc_scatter(h, src_r, dst_r)
        h = _conv_tc(h, p, w, g.reshape(1, C), b.reshape(1, C))

    return _tail_tc(h, Wsl, bsl.reshape(1, C), gamma7.reshape(1, C),
                    beta7.reshape(1, C), W1, b1.reshape(1, C), W2,
                    b2.reshape(1, C), W3, b3.reshape(1, 1))


# speedup vs baseline: 12.0955x; 1.0817x over previous
import functools

import jax
import jax.numpy as jnp
from jax import lax
from jax.experimental import pallas as pl
from jax.experimental.pallas import tpu as pltpu
from jax.experimental.pallas import tpu_sc as plsc

N = 10000
E = 320000
C = 128
EPS = 1e-5

_NC = 2
_NS = 16
_NW = _NC * _NS
_K = 128
_EPT = E // _NW
_NCHUNK = 80
_NPAIR = _NCHUNK // 2
_EPAD = _NCHUNK * _K - _EPT
_NPAD = 10112
_RPT = _NPAD // _NS


def _sc_body(h_hbm, src_hbm, dst_hbm, out_hbm, acc_shared, src_v, dst_ring,
             rows_v, sem0, sem1, semd0, semd1):
    cid = lax.axis_index("c")
    sid = lax.axis_index("s")
    wid = sid * _NC + cid

    ci = pltpu.async_copy(src_hbm.at[wid], src_v, sem0)
    pltpu.async_copy(dst_hbm.at[wid, 0], dst_ring.at[0], semd0)
    pltpu.async_copy(dst_hbm.at[wid, 1], dst_ring.at[1], semd1)

    z = jnp.zeros((16,), jnp.float32)

    def zrow(i, carry):
        for j in range(C // 16):
            rows_v[1, i, pl.ds(j * 16, 16)] = z
        return carry

    lax.fori_loop(0, _K, zrow, 0)
    zcs = [pltpu.async_copy(rows_v.at[1],
                            acc_shared.at[pl.ds(sid * _RPT + t * _K, _K)],
                            sem1)
           for t in range(4)]
    zcs.append(pltpu.async_copy(
        rows_v.at[1, pl.ds(0, _RPT - 4 * _K)],
        acc_shared.at[pl.ds(sid * _RPT + 4 * _K, _RPT - 4 * _K)], sem1))
    ci.wait()
    pltpu.async_copy(h_hbm.at[src_v.at[0]], rows_v.at[0], sem0)
    for c in zcs:
        c.wait()

    plsc.subcore_barrier()

    def pair(t, carry):
        j0 = 2 * t
        j1 = j0 + 1
        pltpu.async_copy(h_hbm.at[src_v.at[j1]], rows_v.at[1], sem1)
        pltpu.make_async_copy(h_hbm.at[src_v.at[j0]], rows_v.at[0],
                              sem0).wait()
        pltpu.make_async_copy(dst_hbm.at[wid, j0], dst_ring.at[0],
                              semd0).wait()
        pltpu.sync_copy(rows_v.at[0], acc_shared.at[dst_ring.at[0]],
                        add=True)

        @pl.when(t + 1 < _NPAIR)
        def _():
            pltpu.async_copy(dst_hbm.at[wid, j0 + 2], dst_ring.at[0], semd0)
            pltpu.async_copy(h_hbm.at[src_v.at[j0 + 2]], rows_v.at[0], sem0)

        pltpu.make_async_copy(h_hbm.at[src_v.at[j1]], rows_v.at[1],
                              sem1).wait()
        pltpu.make_async_copy(dst_hbm.at[wid, j1], dst_ring.at[1],
                              semd1).wait()
        pltpu.sync_copy(rows_v.at[1], acc_shared.at[dst_ring.at[1]],
                        add=True)

        @pl.when(t + 1 < _NPAIR)
        def _():
            pltpu.async_copy(dst_hbm.at[wid, j1 + 2], dst_ring.at[1], semd1)

        return carry

    lax.fori_loop(0, _NPAIR, pair, 0)

    plsc.subcore_barrier()

    pltpu.sync_copy(acc_shared.at[pl.ds(sid * _RPT, _RPT)],
                    out_hbm.at[cid, pl.ds(sid * _RPT, _RPT)])


_sc_scatter = functools.partial(
    pl.kernel,
    _sc_body,
    out_type=jax.ShapeDtypeStruct((_NC, _NPAD, C), jnp.float32),
    mesh=plsc.VectorSubcoreMesh(core_axis_name="c", subcore_axis_name="s"),
    scratch_types=[
        pltpu.VMEM_SHARED((_NPAD, C), jnp.float32),
        pltpu.VMEM((_NCHUNK, _K), jnp.int32),
        pltpu.VMEM((2, _K), jnp.int32),
        pltpu.VMEM((2, _K, C), jnp.float32),
        pltpu.SemaphoreType.DMA,
        pltpu.SemaphoreType.DMA,
        pltpu.SemaphoreType.DMA,
        pltpu.SemaphoreType.DMA,
    ],
)()


_PREC = jax.lax.Precision.DEFAULT


def _dot(a, b):
    return jax.lax.dot_general(a, b, (((1,), (0,)), ((), ())),
                               preferred_element_type=jnp.float32,
                               precision=_PREC)


def _conv_body(x_ref, p_ref, w_ref, g_ref, b_ref, o_ref):
    x = x_ref[...]
    agg = p_ref[0, :N] + p_ref[1, :N]
    h = _dot(x, w_ref[:C]) + _dot(agg, w_ref[C:])
    mu = jnp.mean(h, axis=0, keepdims=True)
    d = h - mu
    var = jnp.mean(d * d, axis=0, keepdims=True)
    o_ref[...] = jnp.maximum(
        d * jax.lax.rsqrt(var + EPS) * g_ref[...] + b_ref[...], 0.0)


def _conv_tc(x, p, w, gamma, beta):
    return pl.pallas_call(
        _conv_body,
        out_shape=jax.ShapeDtypeStruct((N, C), jnp.float32),
    )(x, p, w, gamma, beta)


def _tail_body(x_ref, p_ref, w_ref, g6_ref, b6_ref, wsl_ref, bsl_ref,
               g_ref, b_ref, w1_ref, b1_ref, w2_ref, b2_ref, w3_ref,
               b3_ref, o_ref):
    x = x_ref[...]
    agg = p_ref[0, :N] + p_ref[1, :N]
    hh = _dot(x, w_ref[:C]) + _dot(agg, w_ref[C:])
    mu6 = jnp.mean(hh, axis=0, keepdims=True)
    d6 = hh - mu6
    var6 = jnp.mean(d6 * d6, axis=0, keepdims=True)
    h = jnp.maximum(
        d6 * jax.lax.rsqrt(var6 + EPS) * g6_ref[...] + b6_ref[...], 0.0)
    t = jnp.maximum(_dot(h, wsl_ref[...]) + bsl_ref[...], 0.0)
    mu = jnp.mean(t, axis=0, keepdims=True)
    d = t - mu
    var = jnp.mean(d * d, axis=0, keepdims=True)
    t = jnp.maximum(d * jax.lax.rsqrt(var + EPS) * g_ref[...] + b_ref[...],
                    0.0)
    pm = jnp.mean(t, axis=0, keepdims=True)
    px = jnp.max(t, axis=0, keepdims=True)
    o = jnp.maximum(_dot(pm, w1_ref[:C]) + _dot(px, w1_ref[C:])
                    + b1_ref[...], 0.0)
    o = jnp.maximum(_dot(o, w2_ref[...]) + b2_ref[...], 0.0)
    o_ref[...] = _dot(o, w3_ref[...]) + b3_ref[...]


def _tail_tc(x, p, w6, g6, b6, wsl, bsl, gamma, beta, w1, b1, w2, b2,
             w3, b3):
    return pl.pallas_call(
        _tail_body,
        out_shape=jax.ShapeDtypeStruct((1, 1), jnp.float32),
    )(x, p, w6, g6, b6, wsl, bsl, gamma, beta, w1, b1, w2, b2, w3, b3)


def _pad_edges(edge_index):
    srcs = edge_index[0].reshape(_NW, _EPT)
    dsts = edge_index[1].reshape(_NW, _EPT)
    wids = jnp.arange(_NW, dtype=jnp.int32)[:, None]
    ks = jnp.arange(_EPAD, dtype=jnp.int32)[None, :]
    pad_src = (wids * 313 + ks * 89) % N
    pad_dst = N + (wids * 7 + ks) % (_NPAD - N)
    src_r = jnp.concatenate([srcs, pad_src], axis=1).reshape(
        _NW, _NCHUNK, _K)
    dst_r = jnp.concatenate([dsts, pad_dst], axis=1).reshape(
        _NW, _NCHUNK, _K)
    return src_r, dst_r


def kernel(x, edge_index, Wu1, Wg1, Wg2, Wsu, Wsg, Wst, Wsl, W1, W2, W3,
           gamma1, beta1, gamma2, beta2, gamma3, beta3, gamma4, beta4,
           gamma5, beta5, gamma6, beta6, gamma7, beta7, bsl, b1, b2, b3):
    src_r, dst_r = _pad_edges(edge_index)

    h = x
    layers = ((Wu1, gamma1, beta1), (Wg1, gamma2, beta2),
              (Wg2, gamma3, beta3), (Wsu, gamma4, beta4),
              (Wsg, gamma5, beta5))
    for w, g, b in layers:
        p = _sc_scatter(h, src_r, dst_r)
        h = _conv_tc(h, p, w, g.reshape(1, C), b.reshape(1, C))

    p = _sc_scatter(h, src_r, dst_r)
    return _tail_tc(h, p, Wst, gamma6.reshape(1, C), beta6.reshape(1, C),
                    Wsl, bsl.reshape(1, C), gamma7.reshape(1, C),
                    beta7.reshape(1, C), W1, b1.reshape(1, C), W2,
                    b2.reshape(1, C), W3, b3.reshape(1, 1))
